# baseline (speedup 1.0000x reference)
import functools

import jax
import jax.numpy as jnp
from jax.experimental import pallas as pl

_B, _S, _V, _E, _T = 4, 2048, 1000, 768, 2
_TM = 2048


def _body(x_ref, tt_ref, w_ref, pos_ref, tyt_ref, gb_ref, o_ref, *, s_tiles):
    y = jnp.dot(x_ref[0], w_ref[...], preferred_element_type=jnp.float32)

    s_idx = pl.program_id(0) % s_tiles
    pos = pos_ref[pl.ds(s_idx * _TM, _TM), :]
    tt = tt_ref[0, 0, :].astype(jnp.float32)[:, None]
    ty0 = tyt_ref[0:1, :]
    ty1 = tyt_ref[1:2, :]
    y = y + pos + ty0 + tt * (ty1 - ty0)

    mean = jnp.mean(y, axis=1, keepdims=True)
    yc = y - mean
    var = jnp.mean(yc * yc, axis=1, keepdims=True)
    inv = jax.lax.rsqrt(var + 1e-3)
    o_ref[0] = yc * inv * gb_ref[0:1, :] + gb_ref[1:2, :]


def kernel(input_ids, token_type_ids, token_embedding, position_table, type_table, gamma, beta):
    B, S, V = input_ids.shape
    E = token_embedding.shape[1]
    n_tiles = (B * S) // _TM
    s_tiles = S // _TM

    tt = token_type_ids.reshape(n_tiles, 1, _TM)
    gb = jnp.stack([gamma, beta])

    body = functools.partial(_body, s_tiles=s_tiles)

    out = pl.pallas_call(
        body,
        grid=(n_tiles,),
        in_specs=[
            pl.BlockSpec((1, _TM, V), lambda i, s=s_tiles: (i // s, i % s, 0)),
            pl.BlockSpec((1, 1, _TM), lambda i: (i, 0, 0)),
            pl.BlockSpec((V, E), lambda i: (0, 0)),
            pl.BlockSpec((S, E), lambda i: (0, 0)),
            pl.BlockSpec((_T, E), lambda i: (0, 0)),
            pl.BlockSpec((2, E), lambda i: (0, 0)),
        ],
        out_specs=pl.BlockSpec((1, _TM, E), lambda i, s=s_tiles: (i // s, i % s, 0)),
        out_shape=jax.ShapeDtypeStruct((B, S, E), jnp.float32),
    )(input_ids, tt, token_embedding, position_table, type_table, gb)

    return out


# speedup vs baseline: 1.0228x; 1.0228x over previous
import functools

import jax
import jax.numpy as jnp
from jax.experimental import pallas as pl

_B, _S, _V, _E, _T = 4, 2048, 1000, 768, 2
_TM = 1024
_TH = _TM // 2


def _body(xa_ref, xb_ref, tt_ref, w_ref, pos_ref, tyt_ref, gb_ref, o_ref,
          *, s_tiles):
    w = w_ref[...]
    ty0 = tyt_ref[0:1, :]
    tyd = tyt_ref[1:2, :] - ty0
    gamma = gb_ref[0:1, :]
    beta = gb_ref[1:2, :]
    s_idx = pl.program_id(0) % s_tiles
    tt = tt_ref[0, 0, :].astype(jnp.float32)[:, None]

    for h, x_ref in ((0, xa_ref), (1, xb_ref)):
        y = jnp.dot(x_ref[0], w, preferred_element_type=jnp.float32)
        pos = pos_ref[pl.ds(s_idx * _TM + h * _TH, _TH), :]
        y = y + pos + ty0 + tt[h * _TH:(h + 1) * _TH] * tyd
        mean = jnp.mean(y, axis=1, keepdims=True)
        yc = y - mean
        var = jnp.mean(yc * yc, axis=1, keepdims=True)
        inv = jax.lax.rsqrt(var + 1e-3)
        o_ref[0, h * _TH:(h + 1) * _TH, :] = yc * inv * gamma + beta


def kernel(input_ids, token_type_ids, token_embedding, position_table, type_table, gamma, beta):
    B, S, V = input_ids.shape
    E = token_embedding.shape[1]
    n_tiles = (B * S) // _TM
    s_tiles = S // _TM

    tt = token_type_ids.reshape(n_tiles, 1, _TM)
    gb = jnp.stack([gamma, beta])

    body = functools.partial(_body, s_tiles=s_tiles)

    out = pl.pallas_call(
        body,
        grid=(n_tiles,),
        in_specs=[
            pl.BlockSpec((1, _TH, V), lambda i, s=s_tiles: (i // s, 2 * (i % s), 0)),
            pl.BlockSpec((1, _TH, V), lambda i, s=s_tiles: (i // s, 2 * (i % s) + 1, 0)),
            pl.BlockSpec((1, 1, _TM), lambda i: (i, 0, 0)),
            pl.BlockSpec((V, E), lambda i: (0, 0)),
            pl.BlockSpec((S, E), lambda i: (0, 0)),
            pl.BlockSpec((_T, E), lambda i: (0, 0)),
            pl.BlockSpec((2, E), lambda i: (0, 0)),
        ],
        out_specs=pl.BlockSpec((1, _TM, E), lambda i, s=s_tiles: (i // s, i % s, 0)),
        out_shape=jax.ShapeDtypeStruct((B, S, E), jnp.float32),
    )(input_ids, input_ids, tt, token_embedding, position_table, type_table, gb)

    return out
